# Initial kernel scaffold; baseline (speedup 1.0000x reference)
import jax
import jax.numpy as jnp
from jax.experimental import pallas as pl


def kernel(x, adj_indices, adj_values, W, b, gamma, beta):
    raise NotImplementedError("write your pallas kernel here")


# speedup vs baseline: 3.5618x; 3.5618x over previous
import functools

import jax
import jax.numpy as jnp
from jax import lax
from jax.experimental import pallas as pl
from jax.experimental.pallas import tpu as pltpu
from jax.experimental.pallas import tpu_sc as plsc

N = 10000
D = 128
NC = 2
NS = 16
NW = NC * NS
K = 64
NB = 4
NE = 8
ROWS_PER_TILE = 632
NP = NS * ROWS_PER_TILE


def _sc_body(x_hbm, col_hbm, row_hbm, val_hbm, out_hbm,
             colb, rowb, valb, rows_bufs, agg_sh, esem, gsem, ssem):
    c = lax.axis_index("c")
    s = lax.axis_index("s")
    nchunk = col_hbm.shape[0] // (NW * K)
    tile_base = (c * NS + s) * nchunk * K

    def edge_dma(ch, j):
        base = tile_base + ch * K
        pltpu.async_copy(col_hbm.at[pl.ds(base, K)], colb[j], esem[j])
        pltpu.async_copy(row_hbm.at[pl.ds(base, K)], rowb[j], esem[j])
        pltpu.async_copy(val_hbm.at[pl.ds(base, K)], valb[j], esem[j])

    def edge_wait(j):
        pltpu.make_async_copy(col_hbm.at[pl.ds(0, K)], colb[j], esem[j]).wait()
        pltpu.make_async_copy(row_hbm.at[pl.ds(0, K)], rowb[j], esem[j]).wait()
        pltpu.make_async_copy(val_hbm.at[pl.ds(0, K)], valb[j], esem[j]).wait()

    def gather(je, b):
        pltpu.async_copy(x_hbm.at[colb[je]], rows_bufs[b], gsem[b])

    for ch in range(4):
        edge_dma(ch, ch)
    edge_wait(0)
    gather(0, 0)
    edge_wait(1)
    gather(1, 1)

    zbuf = rows_bufs[NB - 1]
    zv = jnp.zeros((16,), jnp.float32)

    def zrow(i, carry):
        for d in range(D // 16):
            zbuf[i, pl.ds(d * 16, 16)] = zv
        return carry

    lax.fori_loop(0, K, zrow, 0)
    for t in range(ROWS_PER_TILE // K):
        pltpu.sync_copy(zbuf,
                        agg_sh.at[pl.ds(s * ROWS_PER_TILE + t * K, K)])
    _rem = ROWS_PER_TILE % K
    if _rem:
        pltpu.sync_copy(
            zbuf.at[pl.ds(0, _rem)],
            agg_sh.at[pl.ds(s * ROWS_PER_TILE + (ROWS_PER_TILE // K) * K,
                            _rem)])
    plsc.subcore_barrier()

    def scale(je, b):
        rows_v = rows_bufs[b]
        val_v = valb[je]

        def group_body(g, carry):
            v16 = val_v[pl.ds(g * 16, 16)]
            for j in range(16):
                i = g * 16 + j
                vb = jnp.full((16,), v16[j])
                for d in range(D // 16):
                    rows_v[i, pl.ds(d * 16, 16)] = (
                        rows_v[i, pl.ds(d * 16, 16)] * vb)
            return carry

        lax.fori_loop(0, K // 16, group_body, 0)

    def scatter_wait(b):
        pltpu.make_async_copy(rows_bufs[b], agg_sh.at[rowb[0]],
                              ssem[b]).wait()

    def step(ch, k, wait_sc, issue_e, issue_g):
        b = k % NB
        je = k % NE
        bg = (b + 2) % NB
        pltpu.make_async_copy(x_hbm.at[colb[0]], rows_bufs[b],
                              gsem[b]).wait()
        if wait_sc:
            scatter_wait(bg)
        if issue_g:
            edge_wait((k + 2) % NE)
            gather((k + 2) % NE, bg)
        if issue_e:
            edge_dma(ch + 4, (k + 4) % NE)
        scale(je, b)
        pltpu.async_copy(rows_bufs[b], agg_sh.at[rowb[je]], ssem[b],
                         add=True)

    def run_step(ch, k, nch):
        step(ch, k, wait_sc=(ch >= 2), issue_e=(ch + 4 < nch),
             issue_g=(ch + 2 < nch))

    for k in range(NE):
        run_step(k, k, nchunk)

    def it_body(it, carry):
        for k in range(NE):
            step(it * NE + k, k, wait_sc=True, issue_e=True, issue_g=True)
        return carry

    lax.fori_loop(1, nchunk // NE - 1, it_body, 0)

    for k in range(NE):
        run_step(nchunk - NE + k, k, nchunk)

    scatter_wait((nchunk - 2) % NB)
    scatter_wait((nchunk - 1) % NB)

    plsc.subcore_barrier()
    pltpu.sync_copy(agg_sh.at[pl.ds(s * ROWS_PER_TILE, ROWS_PER_TILE)],
                    out_hbm.at[pl.ds(c * NP + s * ROWS_PER_TILE, ROWS_PER_TILE)])


def _make_sc_call(nchunk_total):
    return pl.kernel(
        _sc_body,
        out_type=jax.ShapeDtypeStruct((NC * NP, D), jnp.float32),
        mesh=plsc.VectorSubcoreMesh(core_axis_name="c", subcore_axis_name="s"),
        scratch_types=[
            [pltpu.VMEM((K,), jnp.int32) for _ in range(NE)],
            [pltpu.VMEM((K,), jnp.int32) for _ in range(NE)],
            [pltpu.VMEM((K,), jnp.float32) for _ in range(NE)],
            [pltpu.VMEM((K, D), jnp.float32) for _ in range(NB)],
            pltpu.VMEM_SHARED((NP, D), jnp.float32),
            [pltpu.SemaphoreType.DMA for _ in range(NE)],
            [pltpu.SemaphoreType.DMA for _ in range(NB)],
            [pltpu.SemaphoreType.DMA for _ in range(NB)],
        ],
    )


def _tc_body(p_ref, w_ref, b_ref, g_ref, be_ref, o_ref):
    agg = p_ref[0] + p_ref[1]
    h = lax.dot_general(agg, w_ref[...], (((1,), (1,)), ((), ())),
                        preferred_element_type=jnp.float32,
                        precision=lax.Precision.HIGHEST)
    h = h + b_ref[...]
    mean = jnp.mean(h, axis=-1, keepdims=True)
    center = h - mean
    var = jnp.mean(center * center, axis=-1, keepdims=True)
    hn = center * lax.rsqrt(var + 1e-5) * g_ref[...] + be_ref[...]
    o_ref[...] = jnp.maximum(hn, 0.0)


_TC_BLK = 1000


def _tc_call(partials, W, b, gamma, beta):
    return pl.pallas_call(
        _tc_body,
        grid=(N // _TC_BLK,),
        in_specs=[
            pl.BlockSpec((NC, _TC_BLK, D), lambda i: (0, i, 0)),
            pl.BlockSpec((D, D), lambda i: (0, 0)),
            pl.BlockSpec((1, D), lambda i: (0, 0)),
            pl.BlockSpec((1, D), lambda i: (0, 0)),
            pl.BlockSpec((1, D), lambda i: (0, 0)),
        ],
        out_specs=pl.BlockSpec((_TC_BLK, D), lambda i: (i, 0)),
        out_shape=jax.ShapeDtypeStruct((N, D), jnp.float32),
    )(partials, W, b, gamma, beta)


def kernel(x, adj_indices, adj_values, W, b, gamma, beta):
    e = adj_values.shape[0]
    gran = NW * K * NE
    epad = ((e + gran - 1) // gran) * gran
    pad = epad - e
    row = adj_indices[0].astype(jnp.int32)
    col = adj_indices[1].astype(jnp.int32)
    row_p = jnp.concatenate([row, jnp.zeros((pad,), jnp.int32)])
    col_p = jnp.concatenate([col, jnp.zeros((pad,), jnp.int32)])
    val_p = jnp.concatenate([adj_values, jnp.zeros((pad,), jnp.float32)])
    partial = _make_sc_call(epad // K)(x, col_p, row_p, val_p)
    partial = partial.reshape(NC, NP, D)
    return _tc_call(partial, W, b.reshape(1, D), gamma.reshape(1, D),
                    beta.reshape(1, D))


# speedup vs baseline: 12.1240x; 3.4039x over previous
import functools

import jax
import jax.numpy as jnp
from jax import lax
from jax.experimental import pallas as pl
from jax.experimental.pallas import tpu as pltpu
from jax.experimental.pallas import tpu_sc as plsc

N = 10000
D = 128
NC = 2
NS = 16
NW = NC * NS
K = 64
NB = 4
NE = 8
ROWS_PER_TILE = 632
NP = NS * ROWS_PER_TILE


def _sc_body(x_hbm, col_hbm, row_hbm, val_hbm, out_hbm,
             colb, rowb, valb, rows_bufs, agg_sh, esem, gsem, ssem):
    c = lax.axis_index("c")
    s = lax.axis_index("s")
    nchunk = col_hbm.shape[0] // (NW * K)
    tile_base = (c * NS + s) * nchunk * K

    def edge_dma(ch, j):
        base = tile_base + ch * K
        pltpu.async_copy(col_hbm.at[pl.ds(base, K)], colb[j], esem[j])
        pltpu.async_copy(row_hbm.at[pl.ds(base, K)], rowb[j], esem[j])
        pltpu.async_copy(val_hbm.at[pl.ds(base, K)], valb[j], esem[j])

    def edge_wait(j):
        pltpu.make_async_copy(col_hbm.at[pl.ds(0, K)], colb[j], esem[j]).wait()
        pltpu.make_async_copy(row_hbm.at[pl.ds(0, K)], rowb[j], esem[j]).wait()
        pltpu.make_async_copy(val_hbm.at[pl.ds(0, K)], valb[j], esem[j]).wait()

    def gather(je, b):
        pltpu.async_copy(x_hbm.at[colb[je]], rows_bufs[b], gsem[b])

    for ch in range(4):
        edge_dma(ch, ch)
    edge_wait(0)
    gather(0, 0)
    edge_wait(1)
    gather(1, 1)

    zbuf = rows_bufs[NB - 1]
    zv = jnp.zeros((16,), jnp.float32)

    def zrow(i, carry):
        for d in range(D // 16):
            zbuf[i, pl.ds(d * 16, 16)] = zv
        return carry

    lax.fori_loop(0, K, zrow, 0)
    for t in range(ROWS_PER_TILE // K):
        pltpu.sync_copy(zbuf,
                        agg_sh.at[pl.ds(s * ROWS_PER_TILE + t * K, K)])
    _rem = ROWS_PER_TILE % K
    if _rem:
        pltpu.sync_copy(
            zbuf.at[pl.ds(0, _rem)],
            agg_sh.at[pl.ds(s * ROWS_PER_TILE + (ROWS_PER_TILE // K) * K,
                            _rem)])
    plsc.subcore_barrier()

    def scale(je, b):
        rows_v = rows_bufs[b]
        val_v = valb[je]

        def group_body(g, carry):
            v16 = val_v[pl.ds(g * 16, 16)]
            for j in range(16):
                i = g * 16 + j
                vb = jnp.full((16,), v16[j])
                for d in range(D // 16):
                    rows_v[i, pl.ds(d * 16, 16)] = (
                        rows_v[i, pl.ds(d * 16, 16)] * vb)
            return carry

        lax.fori_loop(0, K // 16, group_body, 0)

    def scatter_wait(b):
        pltpu.make_async_copy(rows_bufs[b], agg_sh.at[rowb[0]],
                              ssem[b]).wait()

    def step(ch, k, wait_sc, issue_e, issue_g):
        b = k % NB
        je = k % NE
        bg = (b + 2) % NB
        pltpu.make_async_copy(x_hbm.at[colb[0]], rows_bufs[b],
                              gsem[b]).wait()
        if wait_sc:
            scatter_wait(bg)
        if issue_g:
            edge_wait((k + 2) % NE)
            gather((k + 2) % NE, bg)
        if issue_e:
            edge_dma(ch + 4, (k + 4) % NE)
        scale(je, b)
        pltpu.async_copy(rows_bufs[b], agg_sh.at[rowb[je]], ssem[b],
                         add=True)

    def run_step(ch, k, nch):
        step(ch, k, wait_sc=(ch >= 2), issue_e=(ch + 4 < nch),
             issue_g=(ch + 2 < nch))

    for k in range(NE):
        run_step(k, k, nchunk)

    def it_body(it, carry):
        for k in range(NE):
            step(it * NE + k, k, wait_sc=True, issue_e=True, issue_g=True)
        return carry

    lax.fori_loop(1, nchunk // NE - 1, it_body, 0)

    for k in range(NE):
        run_step(nchunk - NE + k, k, nchunk)

    scatter_wait((nchunk - 2) % NB)
    scatter_wait((nchunk - 1) % NB)

    plsc.subcore_barrier()
    pltpu.sync_copy(agg_sh.at[pl.ds(s * ROWS_PER_TILE, ROWS_PER_TILE)],
                    out_hbm.at[pl.ds(c * NP + s * ROWS_PER_TILE, ROWS_PER_TILE)])


def _make_sc_call(nchunk_total):
    return pl.kernel(
        _sc_body,
        out_type=jax.ShapeDtypeStruct((NC * NP, D), jnp.float32),
        mesh=plsc.VectorSubcoreMesh(core_axis_name="c", subcore_axis_name="s"),
        scratch_types=[
            [pltpu.VMEM((K,), jnp.int32) for _ in range(NE)],
            [pltpu.VMEM((K,), jnp.int32) for _ in range(NE)],
            [pltpu.VMEM((K,), jnp.float32) for _ in range(NE)],
            [pltpu.VMEM((K, D), jnp.float32) for _ in range(NB)],
            pltpu.VMEM_SHARED((NP, D), jnp.float32),
            [pltpu.SemaphoreType.DMA for _ in range(NE)],
            [pltpu.SemaphoreType.DMA for _ in range(NB)],
            [pltpu.SemaphoreType.DMA for _ in range(NB)],
        ],
    )


def _tc_body(p_ref, w_ref, b_ref, g_ref, be_ref, o_ref):
    agg = p_ref[0] + p_ref[1]
    h = lax.dot_general(agg, w_ref[...], (((1,), (1,)), ((), ())),
                        preferred_element_type=jnp.float32,
                        precision=lax.Precision.HIGHEST)
    h = h + b_ref[...]
    mean = jnp.mean(h, axis=-1, keepdims=True)
    center = h - mean
    var = jnp.mean(center * center, axis=-1, keepdims=True)
    hn = center * lax.rsqrt(var + 1e-5) * g_ref[...] + be_ref[...]
    o_ref[...] = jnp.maximum(hn, 0.0)


_TC_BLK = 1000


def _tc_call(partials, W, b, gamma, beta):
    return pl.pallas_call(
        _tc_body,
        grid=(N // _TC_BLK,),
        in_specs=[
            pl.BlockSpec((NC, _TC_BLK, D), lambda i: (0, i, 0)),
            pl.BlockSpec((D, D), lambda i: (0, 0)),
            pl.BlockSpec((1, D), lambda i: (0, 0)),
            pl.BlockSpec((1, D), lambda i: (0, 0)),
            pl.BlockSpec((1, D), lambda i: (0, 0)),
        ],
        out_specs=pl.BlockSpec((_TC_BLK, D), lambda i: (i, 0)),
        out_shape=jax.ShapeDtypeStruct((N, D), jnp.float32),
    )(partials, W, b, gamma, beta)


def kernel(x, adj_indices, adj_values, W, b, gamma, beta):
    e = adj_values.shape[0]
    gran = NW * K * NE
    epad = ((e + gran - 1) // gran) * gran
    pad = epad - e
    row = adj_indices[0].astype(jnp.int32)
    col = adj_indices[1].astype(jnp.int32)
    spread = jnp.arange(pad, dtype=jnp.int32) % N
    row_p = jnp.concatenate([row, spread])
    col_p = jnp.concatenate([col, spread])
    val_p = jnp.concatenate([adj_values, jnp.zeros((pad,), jnp.float32)])
    partial = _make_sc_call(epad // K)(x, col_p, row_p, val_p)
    partial = partial.reshape(NC, NP, D)
    return _tc_call(partial, W, b.reshape(1, D), gamma.reshape(1, D),
                    beta.reshape(1, D))


# speedup vs baseline: 12.4649x; 1.0281x over previous
import functools

import jax
import jax.numpy as jnp
from jax import lax
from jax.experimental import pallas as pl
from jax.experimental.pallas import tpu as pltpu
from jax.experimental.pallas import tpu_sc as plsc

N = 10000
D = 128
NC = 2
NS = 16
NW = NC * NS
K = 64
NB = 5
NE = 10
ROWS_PER_TILE = 632
NP = NS * ROWS_PER_TILE


def _sc_body(x_hbm, col_hbm, row_hbm, val_hbm, out_hbm,
             colb, rowb, valb, rows_bufs, agg_sh, esem, gsem, ssem):
    c = lax.axis_index("c")
    s = lax.axis_index("s")
    nchunk = col_hbm.shape[0] // (NW * K)
    tile_base = (c * NS + s) * nchunk * K

    def edge_dma(ch, j):
        base = tile_base + ch * K
        pltpu.async_copy(col_hbm.at[pl.ds(base, K)], colb[j], esem[j])
        pltpu.async_copy(row_hbm.at[pl.ds(base, K)], rowb[j], esem[j])
        pltpu.async_copy(val_hbm.at[pl.ds(base, K)], valb[j], esem[j])

    def edge_wait(j):
        pltpu.make_async_copy(col_hbm.at[pl.ds(0, K)], colb[j], esem[j]).wait()
        pltpu.make_async_copy(row_hbm.at[pl.ds(0, K)], rowb[j], esem[j]).wait()
        pltpu.make_async_copy(val_hbm.at[pl.ds(0, K)], valb[j], esem[j]).wait()

    def gather(je, b):
        pltpu.async_copy(x_hbm.at[colb[je]], rows_bufs[b], gsem[b])

    for ch in range(6):
        edge_dma(ch, ch)
    edge_wait(0)
    gather(0, 0)
    edge_wait(1)
    gather(1, 1)
    edge_wait(2)
    gather(2, 2)

    zbuf = rows_bufs[NB - 1]
    zv = jnp.zeros((16,), jnp.float32)

    def zrow(i, carry):
        for d in range(D // 16):
            zbuf[i, pl.ds(d * 16, 16)] = zv
        return carry

    lax.fori_loop(0, K, zrow, 0)
    for t in range(ROWS_PER_TILE // K):
        pltpu.sync_copy(zbuf,
                        agg_sh.at[pl.ds(s * ROWS_PER_TILE + t * K, K)])
    _rem = ROWS_PER_TILE % K
    if _rem:
        pltpu.sync_copy(
            zbuf.at[pl.ds(0, _rem)],
            agg_sh.at[pl.ds(s * ROWS_PER_TILE + (ROWS_PER_TILE // K) * K,
                            _rem)])
    plsc.subcore_barrier()

    def scale(je, b):
        rows_v = rows_bufs[b]
        val_v = valb[je]

        def group_body(g, carry):
            v16 = val_v[pl.ds(g * 16, 16)]
            for j in range(16):
                i = g * 16 + j
                vb = jnp.full((16,), v16[j])
                for d in range(D // 16):
                    rows_v[i, pl.ds(d * 16, 16)] = (
                        rows_v[i, pl.ds(d * 16, 16)] * vb)
            return carry

        lax.fori_loop(0, K // 16, group_body, 0)

    def scatter_wait(b):
        pltpu.make_async_copy(rows_bufs[b], agg_sh.at[rowb[0]],
                              ssem[b]).wait()

    def step(ch, k, wait_sc, issue_e, issue_g):
        b = k % NB
        je = k % NE
        bg = (b + 3) % NB
        pltpu.make_async_copy(x_hbm.at[colb[0]], rows_bufs[b],
                              gsem[b]).wait()
        if wait_sc:
            scatter_wait(bg)
        if issue_g:
            edge_wait((k + 3) % NE)
            gather((k + 3) % NE, bg)
        if issue_e:
            edge_dma(ch + 6, (k + 6) % NE)
        scale(je, b)
        pltpu.async_copy(rows_bufs[b], agg_sh.at[rowb[je]], ssem[b],
                         add=True)

    def run_step(ch, k, nch):
        step(ch, k, wait_sc=(ch >= 2), issue_e=(ch + 6 < nch),
             issue_g=(ch + 3 < nch))

    for k in range(NE):
        run_step(k, k, nchunk)

    def it_body(it, carry):
        for k in range(NE):
            step(it * NE + k, k, wait_sc=True, issue_e=True, issue_g=True)
        return carry

    lax.fori_loop(1, nchunk // NE - 1, it_body, 0)

    for k in range(NE):
        run_step(nchunk - NE + k, k, nchunk)

    scatter_wait((nchunk - 2) % NB)
    scatter_wait((nchunk - 1) % NB)

    plsc.subcore_barrier()
    pltpu.sync_copy(agg_sh.at[pl.ds(s * ROWS_PER_TILE, ROWS_PER_TILE)],
                    out_hbm.at[pl.ds(c * NP + s * ROWS_PER_TILE, ROWS_PER_TILE)])


def _make_sc_call(nchunk_total):
    return pl.kernel(
        _sc_body,
        out_type=jax.ShapeDtypeStruct((NC * NP, D), jnp.float32),
        mesh=plsc.VectorSubcoreMesh(core_axis_name="c", subcore_axis_name="s"),
        scratch_types=[
            [pltpu.VMEM((K,), jnp.int32) for _ in range(NE)],
            [pltpu.VMEM((K,), jnp.int32) for _ in range(NE)],
            [pltpu.VMEM((K,), jnp.float32) for _ in range(NE)],
            [pltpu.VMEM((K, D), jnp.float32) for _ in range(NB)],
            pltpu.VMEM_SHARED((NP, D), jnp.float32),
            [pltpu.SemaphoreType.DMA for _ in range(NE)],
            [pltpu.SemaphoreType.DMA for _ in range(NB)],
            [pltpu.SemaphoreType.DMA for _ in range(NB)],
        ],
    )


def _tc_body(p_ref, w_ref, b_ref, g_ref, be_ref, o_ref):
    agg = p_ref[0] + p_ref[1]
    h = lax.dot_general(agg, w_ref[...], (((1,), (1,)), ((), ())),
                        preferred_element_type=jnp.float32,
                        precision=lax.Precision.HIGHEST)
    h = h + b_ref[...]
    mean = jnp.mean(h, axis=-1, keepdims=True)
    center = h - mean
    var = jnp.mean(center * center, axis=-1, keepdims=True)
    hn = center * lax.rsqrt(var + 1e-5) * g_ref[...] + be_ref[...]
    o_ref[...] = jnp.maximum(hn, 0.0)


_TC_BLK = 1000


def _tc_call(partials, W, b, gamma, beta):
    return pl.pallas_call(
        _tc_body,
        grid=(N // _TC_BLK,),
        in_specs=[
            pl.BlockSpec((NC, _TC_BLK, D), lambda i: (0, i, 0)),
            pl.BlockSpec((D, D), lambda i: (0, 0)),
            pl.BlockSpec((1, D), lambda i: (0, 0)),
            pl.BlockSpec((1, D), lambda i: (0, 0)),
            pl.BlockSpec((1, D), lambda i: (0, 0)),
        ],
        out_specs=pl.BlockSpec((_TC_BLK, D), lambda i: (i, 0)),
        out_shape=jax.ShapeDtypeStruct((N, D), jnp.float32),
    )(partials, W, b, gamma, beta)


def kernel(x, adj_indices, adj_values, W, b, gamma, beta):
    e = adj_values.shape[0]
    gran = NW * K * NE
    epad = ((e + gran - 1) // gran) * gran
    pad = epad - e
    row = adj_indices[0].astype(jnp.int32)
    col = adj_indices[1].astype(jnp.int32)
    spread = jnp.arange(pad, dtype=jnp.int32) % N
    row_p = jnp.concatenate([row, spread])
    col_p = jnp.concatenate([col, spread])
    val_p = jnp.concatenate([adj_values, jnp.zeros((pad,), jnp.float32)])
    partial = _make_sc_call(epad // K)(x, col_p, row_p, val_p)
    partial = partial.reshape(NC, NP, D)
    return _tc_call(partial, W, b.reshape(1, D), gamma.reshape(1, D),
                    beta.reshape(1, D))
